# baseline (speedup 1.0000x reference)
import functools

import jax
import jax.numpy as jnp
import numpy as np
from jax import lax
from jax.experimental import pallas as pl
from jax.experimental.pallas import tpu as pltpu, tpu_sc as plsc

_B, _N, _C = 16, 512, 256
_T = 2048
_HALF = _T // 2
_CHUNK = 128
_NCHUNK = _HALF // _CHUNK
_NPAD = 64
_ZERO_ROW = _B * _N


def _lr_body(table, durf, zero_hbm, out_hbm, mask_hbm,
             dur_v, a_v, idx_v, msk_v, rows0_v, rows1_v, zbuf_v,
             sem0, sem1, semz, wsem0, wsem1, wsemz):
    c = lax.axis_index("c")
    s = lax.axis_index("s")
    wid = c * 16 + s
    i = wid // 2
    h = wid % 2
    t0 = h * _HALF

    zcp = pltpu.async_copy(zero_hbm, zbuf_v, semz)
    pltpu.sync_copy(durf.at[i], dur_v)

    zeros16 = jnp.zeros((16,), jnp.int32)
    zeros16f = jnp.zeros((16,), jnp.float32)
    iota16 = lax.broadcasted_iota(jnp.int32, (16,), 0)

    def z_body(j, carry):
        a_v[pl.ds(j * 16, 16)] = zeros16
        return carry

    lax.fori_loop(0, _HALF // 16, z_body, 0)

    def cs_body(j, carry):
        tot, base = carry
        v = jnp.maximum(dur_v[pl.ds(j * 16, 16)], 0)
        cs = plsc.cumsum(v) + tot
        ex = cs - v
        n_vec = j * 16 + iota16
        m = (v > 0) & (ex >= t0) & (ex < t0 + _HALF)
        plsc.store_scatter(a_v, [ex - t0], n_vec, mask=m)
        base = base + jnp.sum((cs <= t0).astype(jnp.int32))
        return (jnp.max(cs), base)

    tot, base = lax.fori_loop(0, _N // 16, cs_body,
                              (jnp.int32(0), jnp.int32(0)))

    def cm_body(j, run):
        a = a_v[pl.ds(j * 16, 16)]
        cm = jnp.maximum(plsc.cummax(a), run)
        t_vec = t0 + j * 16 + iota16
        valid = t_vec < tot
        idx_v[pl.ds(j * 16, 16)] = i * _N + jnp.where(
            valid, cm, t_vec & (_N - 1))
        msk_v[pl.ds(j * 16, 16)] = valid.astype(jnp.int32)
        return jnp.max(cm)

    out_base = i * _T + t0
    vrows = jnp.clip(tot - t0, 0, _HALF)

    def dst_of(jj):
        return out_hbm.at[i, pl.ds(t0 + jj * _CHUNK, _CHUNK)]

    def gather_src(jj):
        return table.at[idx_v.at[pl.ds(jj * _CHUNK, _CHUNK)]]

    zcp.wait()

    def zfire(jj, carry):
        @pl.when(jj * _CHUNK >= vrows)
        def _():
            pltpu.async_copy(zbuf_v, dst_of(jj), wsemz)
        return carry

    lax.fori_loop(0, _NCHUNK, zfire, 0)

    def g_issue(jj, buf, sem):
        @pl.when(jj * _CHUNK < vrows)
        def _():
            pltpu.async_copy(gather_src(jj), buf, sem)

    def g_drain(jj, buf, sem, wsem):
        @pl.when(jj * _CHUNK < vrows)
        def _():
            pltpu.make_async_copy(gather_src(jj), buf, sem).wait()

            @pl.when(vrows < (jj + 1) * _CHUNK)
            def _():
                kk = vrows - jj * _CHUNK
                up8 = jnp.minimum(((kk + 7) >> 3) << 3, _CHUNK)

                def zrow(r, carry):
                    for cpart in range(_C // 16):
                        buf[r, pl.ds(cpart * 16, 16)] = zeros16f
                    return carry

                lax.fori_loop(kk, up8, zrow, 0)
                m8 = _CHUNK - up8
                pos = up8
                for b in (64, 32, 16, 8):
                    hit = (m8 & b) != 0

                    @pl.when(hit)
                    def _(b=b, pos=pos):
                        pltpu.sync_copy(
                            zero_hbm.at[pl.ds(0, b)],
                            buf.at[pl.ds(pl.multiple_of(pos, 8), b)])

                    pos = pos + jnp.where(hit, b, 0)

            pltpu.async_copy(buf, dst_of(jj), wsem)

    _CV = _CHUNK // 16
    run = lax.fori_loop(0, _CV, cm_body, base)
    g_issue(0, rows0_v, sem0)
    run = lax.fori_loop(_CV, 2 * _CV, cm_body, run)
    g_issue(1, rows1_v, sem1)
    lax.fori_loop(2 * _CV, _HALF // 16, cm_body, run)
    pltpu.sync_copy(msk_v, mask_hbm.at[i, pl.ds(t0, _HALF)])

    def ring(p, carry):
        j0 = 2 * p
        g_drain(j0, rows0_v, sem0, wsem0)

        @pl.when((j0 + 2) * _CHUNK < vrows)
        def _():
            pltpu.make_async_copy(rows0_v, dst_of(j0), wsem0).wait()
            pltpu.async_copy(gather_src(j0 + 2), rows0_v, sem0)

        g_drain(j0 + 1, rows1_v, sem1, wsem1)

        @pl.when((j0 + 3) * _CHUNK < vrows)
        def _():
            pltpu.make_async_copy(rows1_v, dst_of(j0 + 1), wsem1).wait()
            pltpu.async_copy(gather_src(j0 + 3), rows1_v, sem1)

        return carry

    lax.fori_loop(0, _NCHUNK // 2, ring, 0)

    def wdrain(p, carry):
        for off, buf, wsem in ((0, rows0_v, wsem0), (1, rows1_v, wsem1)):
            jj = 2 * p + off

            @pl.when((jj * _CHUNK < vrows) & ((jj + 2) * _CHUNK >= vrows))
            def _(jj=jj, buf=buf, wsem=wsem):
                pltpu.make_async_copy(buf, dst_of(jj), wsem).wait()

            @pl.when(jj * _CHUNK >= vrows)
            def _(jj=jj):
                pltpu.make_async_copy(zbuf_v, dst_of(jj), wsemz).wait()
        return carry

    lax.fori_loop(0, _NCHUNK // 2, wdrain, 0)


_lr_kernel = functools.partial(
    pl.kernel,
    mesh=plsc.VectorSubcoreMesh(core_axis_name="c", subcore_axis_name="s"),
    compiler_params=pltpu.CompilerParams(needs_layout_passes=False),
    out_type=(jax.ShapeDtypeStruct((_B, _T, _C), jnp.float32),
              jax.ShapeDtypeStruct((_B, _T), jnp.int32)),
    scratch_types=[
        pltpu.VMEM((_N,), jnp.int32),
        pltpu.VMEM((_HALF,), jnp.int32),
        pltpu.VMEM((_HALF,), jnp.int32),
        pltpu.VMEM((_HALF,), jnp.int32),
        pltpu.VMEM((_CHUNK, _C), jnp.float32),
        pltpu.VMEM((_CHUNK, _C), jnp.float32),
        pltpu.VMEM((_CHUNK, _C), jnp.float32),
        pltpu.SemaphoreType.DMA,
        pltpu.SemaphoreType.DMA,
        pltpu.SemaphoreType.DMA,
        pltpu.SemaphoreType.DMA,
        pltpu.SemaphoreType.DMA,
        pltpu.SemaphoreType.DMA,
    ],
)(_lr_body)


_ZERO_BLOCK = np.zeros((_CHUNK, _C), np.float32)


def kernel(x, duration, max_length):
    B, N, C = x.shape
    table = x.reshape(B * N, C)
    out, mask_i32 = _lr_kernel(table, duration, _ZERO_BLOCK)
    return (out, mask_i32 != 0)


# speedup vs baseline: 1.0322x; 1.0322x over previous
import functools

import jax
import jax.numpy as jnp
import numpy as np
from jax import lax
from jax.experimental import pallas as pl
from jax.experimental.pallas import tpu as pltpu, tpu_sc as plsc

_B, _N, _C = 16, 512, 256
_T = 2048
_HALF = _T // 2
_CHUNK = 128
_NCHUNK = _HALF // _CHUNK
_NPAD = 64
_ZERO_ROW = _B * _N


def _lr_body(table, durf, zero_hbm, out_hbm, mask_hbm,
             dur_v, a_v, idx_v, msk_v, rows0_v, rows1_v, zbuf_v,
             sem0, sem1, semz, wsem0, wsem1, wsemz):
    c = lax.axis_index("c")
    s = lax.axis_index("s")
    wid = c * 16 + s
    i = wid // 2
    h = wid % 2
    t0 = h * _HALF

    zcp = pltpu.async_copy(zero_hbm, zbuf_v, semz)
    pltpu.sync_copy(durf.at[i], dur_v)

    zeros16 = jnp.zeros((16,), jnp.int32)
    zeros16f = jnp.zeros((16,), jnp.float32)
    iota16 = lax.broadcasted_iota(jnp.int32, (16,), 0)

    def z_body(j, carry):
        a_v[pl.ds(j * 16, 16)] = zeros16
        return carry

    lax.fori_loop(0, _HALF // 16, z_body, 0)

    def cs_body(j, carry):
        tot, base = carry
        v = jnp.maximum(dur_v[pl.ds(j * 16, 16)], 0)
        cs = plsc.cumsum(v) + tot
        ex = cs - v
        n_vec = j * 16 + iota16
        m = (v > 0) & (ex >= t0) & (ex < t0 + _HALF)
        plsc.store_scatter(a_v, [ex - t0], n_vec, mask=m)
        base = base + jnp.sum((cs <= t0).astype(jnp.int32))
        return (jnp.max(cs), base)

    tot, base = lax.fori_loop(0, _N // 16, cs_body,
                              (jnp.int32(0), jnp.int32(0)))

    def cm_body(j, run):
        a = a_v[pl.ds(j * 16, 16)]
        cm = jnp.maximum(plsc.cummax(a), run)
        t_vec = t0 + j * 16 + iota16
        valid = t_vec < tot
        idx_v[pl.ds(j * 16, 16)] = i * _N + jnp.where(
            valid, cm, t_vec & (_N - 1))
        msk_v[pl.ds(j * 16, 16)] = valid.astype(jnp.int32)
        return jnp.max(cm)

    out_base = i * _T + t0
    vrows = jnp.clip(tot - t0, 0, _HALF)

    def dst_of(jj):
        return out_hbm.at[i, pl.ds(t0 + jj * _CHUNK, _CHUNK)]

    def gather_src(jj):
        return table.at[idx_v.at[pl.ds(jj * _CHUNK, _CHUNK)]]

    def g_issue(jj, buf, sem):
        @pl.when(jj * _CHUNK < vrows)
        def _():
            pltpu.async_copy(gather_src(jj), buf, sem)

    def g_drain(jj, buf, sem, wsem):
        @pl.when(jj * _CHUNK < vrows)
        def _():
            pltpu.make_async_copy(gather_src(jj), buf, sem).wait()

            @pl.when(vrows < (jj + 1) * _CHUNK)
            def _():
                kk = vrows - jj * _CHUNK
                up8 = jnp.minimum(((kk + 7) >> 3) << 3, _CHUNK)

                def zrow(r, carry):
                    for cpart in range(_C // 16):
                        buf[r, pl.ds(cpart * 16, 16)] = zeros16f
                    return carry

                lax.fori_loop(kk, up8, zrow, 0)
                m8 = _CHUNK - up8
                pos = up8
                for b in (64, 32, 16, 8):
                    hit = (m8 & b) != 0

                    @pl.when(hit)
                    def _(b=b, pos=pos):
                        pltpu.sync_copy(
                            zero_hbm.at[pl.ds(0, b)],
                            buf.at[pl.ds(pl.multiple_of(pos, 8), b)])

                    pos = pos + jnp.where(hit, b, 0)

            pltpu.async_copy(buf, dst_of(jj), wsem)

    _CV = _CHUNK // 16
    run = lax.fori_loop(0, _CV, cm_body, base)
    g_issue(0, rows0_v, sem0)
    run = lax.fori_loop(_CV, 2 * _CV, cm_body, run)
    g_issue(1, rows1_v, sem1)

    zcp.wait()

    def zfire(jj, carry):
        @pl.when(jj * _CHUNK >= vrows)
        def _():
            pltpu.async_copy(zbuf_v, dst_of(jj), wsemz)
        return carry

    lax.fori_loop(0, _NCHUNK, zfire, 0)
    lax.fori_loop(2 * _CV, _HALF // 16, cm_body, run)
    pltpu.sync_copy(msk_v, mask_hbm.at[i, pl.ds(t0, _HALF)])

    def ring(p, carry):
        j0 = 2 * p
        g_drain(j0, rows0_v, sem0, wsem0)

        @pl.when((j0 + 2) * _CHUNK < vrows)
        def _():
            pltpu.make_async_copy(rows0_v, dst_of(j0), wsem0).wait()
            pltpu.async_copy(gather_src(j0 + 2), rows0_v, sem0)

        g_drain(j0 + 1, rows1_v, sem1, wsem1)

        @pl.when((j0 + 3) * _CHUNK < vrows)
        def _():
            pltpu.make_async_copy(rows1_v, dst_of(j0 + 1), wsem1).wait()
            pltpu.async_copy(gather_src(j0 + 3), rows1_v, sem1)

        return carry

    lax.fori_loop(0, _NCHUNK // 2, ring, 0)

    def wdrain(p, carry):
        for off, buf, wsem in ((0, rows0_v, wsem0), (1, rows1_v, wsem1)):
            jj = 2 * p + off

            @pl.when((jj * _CHUNK < vrows) & ((jj + 2) * _CHUNK >= vrows))
            def _(jj=jj, buf=buf, wsem=wsem):
                pltpu.make_async_copy(buf, dst_of(jj), wsem).wait()

            @pl.when(jj * _CHUNK >= vrows)
            def _(jj=jj):
                pltpu.make_async_copy(zbuf_v, dst_of(jj), wsemz).wait()
        return carry

    lax.fori_loop(0, _NCHUNK // 2, wdrain, 0)


_lr_kernel = functools.partial(
    pl.kernel,
    mesh=plsc.VectorSubcoreMesh(core_axis_name="c", subcore_axis_name="s"),
    compiler_params=pltpu.CompilerParams(needs_layout_passes=False),
    out_type=(jax.ShapeDtypeStruct((_B, _T, _C), jnp.float32),
              jax.ShapeDtypeStruct((_B, _T), jnp.int32)),
    scratch_types=[
        pltpu.VMEM((_N,), jnp.int32),
        pltpu.VMEM((_HALF,), jnp.int32),
        pltpu.VMEM((_HALF,), jnp.int32),
        pltpu.VMEM((_HALF,), jnp.int32),
        pltpu.VMEM((_CHUNK, _C), jnp.float32),
        pltpu.VMEM((_CHUNK, _C), jnp.float32),
        pltpu.VMEM((_CHUNK, _C), jnp.float32),
        pltpu.SemaphoreType.DMA,
        pltpu.SemaphoreType.DMA,
        pltpu.SemaphoreType.DMA,
        pltpu.SemaphoreType.DMA,
        pltpu.SemaphoreType.DMA,
        pltpu.SemaphoreType.DMA,
    ],
)(_lr_body)


_ZERO_BLOCK = np.zeros((_CHUNK, _C), np.float32)


def kernel(x, duration, max_length):
    B, N, C = x.shape
    table = x.reshape(B * N, C)
    out, mask_i32 = _lr_kernel(table, duration, _ZERO_BLOCK)
    return (out, mask_i32 != 0)
